# baseline (speedup 1.0000x reference)
import jax
import jax.numpy as jnp
from jax import lax
from jax.experimental import pallas as pl
from jax.experimental.pallas import tpu as pltpu
from jax.experimental.pallas import tpu_sc as plsc

N = 10000
E = 320000
D_IN = 16
D_EDGE_ = 4
H = 64
L = 6
F = D_IN + H * (L + 1)

NC, NS = 2, 16
NW = NC * NS
C = 128
NCHUNK = 80
EPW = NCHUNK * C
E_PAD = NW * EPW
NP = 10112
RZ = NP // NS
RO = 632
RO_LAST = N - 15 * RO
DW = 16


SUP = C
NCH_TOT = E_PAD // C
K0 = 128
K1 = 32


def _sc_layer_body(h_hbm, e_hbm, src_hbm, dst_hbm, out_hbm,
                   src_all, dst_all, gbuf, ebuf, sbuf,
                   sem_g0, sem_g1, sem_e0, sem_e1, sem_s0, sem_s1, acc):
    cid = lax.axis_index("c")
    sid = lax.axis_index("s")
    sem_g = (sem_g0, sem_g1)
    sem_e = (sem_e0, sem_e1)
    sem_s = (sem_s0, sem_s1)
    gbase = jnp.where(cid == 0, sid * K0, NS * K0 + sid * K1)
    nk = jnp.where(cid == 0, K0, K1)

    @pl.when(cid == 0)
    def _stage0():
        pltpu.sync_copy(src_hbm.at[pl.ds(sid * K0, K0)],
                        src_all.at[pl.ds(0, K0)])
        pltpu.sync_copy(dst_hbm.at[pl.ds(sid * K0, K0)],
                        dst_all.at[pl.ds(0, K0)])

    @pl.when(cid == 1)
    def _stage1():
        pltpu.sync_copy(src_hbm.at[pl.ds(NS * K0 + sid * K1, K1)],
                        src_all.at[pl.ds(0, K1)])
        pltpu.sync_copy(dst_hbm.at[pl.ds(NS * K0 + sid * K1, K1)],
                        dst_all.at[pl.ds(0, K1)])

    @plsc.parallel_loop(0, SUP, unroll=8)
    def _zrow(r):
        for c4 in range(4):
            sbuf[0, r, pl.ds(c4 * 16, 16)] = jnp.zeros((16,), jnp.float32)

    for i in range(4):
        pltpu.sync_copy(sbuf.at[0],
                        acc.at[pl.ds(sid * RZ + i * SUP, SUP)])
    pltpu.sync_copy(sbuf.at[0, pl.ds(0, RZ - 4 * SUP)],
                    acc.at[pl.ds(sid * RZ + 4 * SUP, RZ - 4 * SUP)])
    plsc.subcore_barrier()

    def issue_loads(k, b):
        pltpu.async_copy(h_hbm.at[src_all.at[k]], gbuf.at[b], sem_g[b])
        pltpu.async_copy(e_hbm.at[pl.ds((gbase + k) * (C // 2), C // 2)],
                         ebuf.at[b], sem_e[b])

    def wait_f32(sem, dst):
        pltpu.make_async_copy(h_hbm.at[pl.ds(0, SUP)], dst, sem).wait()

    def wait_e(sem, dst):
        pltpu.make_async_copy(e_hbm.at[pl.ds(0, SUP // 2)], dst, sem).wait()

    def do_superchunk(k, b):
        @pl.when(k + 1 < nk)
        def _pre():
            issue_loads(k + 1, 1 - b)

        @pl.when(k >= 2)
        def _drain():
            wait_f32(sem_s[b], sbuf.at[b])

        wait_f32(sem_g[b], gbuf.at[b])
        wait_e(sem_e[b], ebuf.at[b])

        @plsc.parallel_loop(0, SUP // 2, unroll=4)
        def _row(r2):
            for c8 in range(8):
                r = 2 * r2 + c8 // 4
                sl = pl.ds((c8 % 4) * 16, 16)
                sbuf[b, r, sl] = jnp.maximum(
                    gbuf[b, r, sl] + ebuf[b, r2, pl.ds(c8 * 16, 16)], 0.0)

        pltpu.async_copy(sbuf.at[b], acc.at[dst_all.at[k]], sem_s[b],
                         add=True)

    issue_loads(0, 0)

    def pair(j, carry):
        do_superchunk(2 * j, 0)
        do_superchunk(2 * j + 1, 1)
        return carry

    lax.fori_loop(0, nk // 2, pair, 0)
    wait_f32(sem_s[0], sbuf.at[0])
    wait_f32(sem_s[1], sbuf.at[1])
    plsc.subcore_barrier()

    @pl.when(sid < NS - 1)
    def _full():
        pltpu.sync_copy(acc.at[pl.ds(sid * RO, RO)],
                        out_hbm.at[pl.ds(sid * RO, RO), pl.ds(cid * H, H)])

    @pl.when(sid == NS - 1)
    def _last():
        pltpu.sync_copy(
            acc.at[pl.ds((NS - 1) * RO, RO_LAST)],
            out_hbm.at[pl.ds((NS - 1) * RO, RO_LAST), pl.ds(cid * H, H)])


_sc_layer = pl.kernel(
    _sc_layer_body,
    out_type=jax.ShapeDtypeStruct((N, 2 * H), jnp.float32),
    mesh=plsc.VectorSubcoreMesh(core_axis_name="c", subcore_axis_name="s"),
    scratch_types=[
        pltpu.VMEM((K0, C), jnp.int32),
        pltpu.VMEM((K0, C), jnp.int32),
        pltpu.VMEM((2, SUP, H), jnp.float32),
        pltpu.VMEM((2, SUP // 2, 2 * H), jnp.float32),
        pltpu.VMEM((2, SUP, H), jnp.float32),
        pltpu.SemaphoreType.DMA,
        pltpu.SemaphoreType.DMA,
        pltpu.SemaphoreType.DMA,
        pltpu.SemaphoreType.DMA,
        pltpu.SemaphoreType.DMA,
        pltpu.SemaphoreType.DMA,
        pltpu.VMEM_SHARED((NP, H), jnp.float32),
    ],
    compiler_params=pltpu.CompilerParams(use_tc_tiling_on_sc=False),
)


def _sc_deg_body(dst_hbm, zero_hbm, out_hbm, dst_all, obuf, acc):
    cid = lax.axis_index("c")
    sid = lax.axis_index("s")
    wid = sid * NC + cid
    pltpu.sync_copy(dst_hbm.at[pl.ds(wid * NCHUNK, NCHUNK)], dst_all)
    pltpu.sync_copy(zero_hbm.at[pl.ds(sid * RZ, RZ)],
                    acc.at[pl.ds(sid * RZ, RZ)])

    def orow(r, carry):
        obuf[r, pl.ds(0, 16)] = jnp.full((16,), 1.0, jnp.float32)
        return carry
    lax.fori_loop(0, C, orow, 0)
    plsc.subcore_barrier()

    def chunk(k, carry):
        pltpu.sync_copy(obuf, acc.at[dst_all.at[k]], add=True)
        return carry
    lax.fori_loop(0, NCHUNK, chunk, 0)
    plsc.subcore_barrier()

    @pl.when(sid < NS - 1)
    def _full():
        pltpu.sync_copy(acc.at[pl.ds(sid * RO, RO)],
                        out_hbm.at[pl.ds(cid * N + sid * RO, RO)])

    @pl.when(sid == NS - 1)
    def _last():
        pltpu.sync_copy(acc.at[pl.ds((NS - 1) * RO, RO_LAST)],
                        out_hbm.at[pl.ds(cid * N + (NS - 1) * RO, RO_LAST)])


_sc_deg = pl.kernel(
    _sc_deg_body,
    out_type=jax.ShapeDtypeStruct((2 * N, DW), jnp.float32),
    mesh=plsc.VectorSubcoreMesh(core_axis_name="c", subcore_axis_name="s"),
    scratch_types=[
        pltpu.VMEM((NCHUNK, C), jnp.int32),
        pltpu.VMEM((C, DW), jnp.float32),
        pltpu.VMEM_SHARED((NP, DW), jnp.float32),
    ],
    compiler_params=pltpu.CompilerParams(use_tc_tiling_on_sc=False),
)


EBLK = 4096
NBLK = 2000


def _e_kernel(attr_ref, we_ref, be_ref, out_ref):
    acc = jnp.broadcast_to(be_ref[...], (EBLK, 2 * H))
    for k in range(2 * D_EDGE_):
        acc = acc + attr_ref[:, k:k + 1] * we_ref[k:k + 1, :]
    out_ref[...] = acc


def _pre_kernel(x_ref, d0_ref, d1_ref, win_ref, bin_ref, bmsg_ref,
                xm_ref, h0_ref, degb_ref):
    col = lax.broadcasted_iota(jnp.int32, (NBLK, D_IN), 1)
    keep = (col < 2) | ((col >= 4) & (col < 10))
    xm = jnp.where(keep, x_ref[...], 0.0)
    xm_ref[...] = xm
    h0_ref[...] = xm @ win_ref[...] + bin_ref[...]
    deg = d0_ref[:, 0:1] + d1_ref[:, 0:1]
    degb_ref[...] = deg * bmsg_ref[...]


def _update_kernel(s_ref, h_ref, degb_ref, w_ref, out_ref):
    out_ref[...] = h_ref[...] + \
        (s_ref[:, :H] + s_ref[:, H:]) @ w_ref[...] + degb_ref[...]


def _max_kernel(xm_ref, h0, h1, h2, h3, h4, h5, h6, out_ref):
    i = pl.program_id(0)

    @pl.when(i == 0)
    def _init():
        out_ref[...] = jnp.full_like(out_ref, -jnp.inf)

    hs = [h0, h1, h2, h3, h4, h5, h6]
    out_ref[:, 0:D_IN] = jnp.maximum(
        out_ref[:, 0:D_IN], jnp.max(xm_ref[...], axis=0, keepdims=True))
    for j, hr in enumerate(hs):
        sl = slice(D_IN + j * H, D_IN + (j + 1) * H)
        out_ref[:, sl] = jnp.maximum(
            out_ref[:, sl], jnp.max(hr[...], axis=0, keepdims=True))


def _assemble_kernel(xm_ref, h0, h1, h2, h3, h4, h5, h6, ge_ref, out_ref):
    hs = [h0, h1, h2, h3, h4, h5, h6]
    out_ref[:, 0:D_IN] = xm_ref[...]
    for j, hr in enumerate(hs):
        out_ref[:, D_IN + j * H:D_IN + (j + 1) * H] = hr[...]
    out_ref[:, F:] = jnp.broadcast_to(ge_ref[...], (NBLK, F))


def kernel(x, edge_index, edge_attr, W_edge, b_edge, W_in, b_in, W_msg, b_msg):
    src = edge_index[0]
    dst = edge_index[1]
    src_p = jnp.pad(src, (0, E_PAD - E)).reshape(NCH_TOT, C)
    dst_p = jnp.pad(dst, (0, E_PAD - E),
                    constant_values=N).reshape(NCH_TOT, C)
    zeros_d = jnp.zeros((NP, DW), jnp.float32)

    bi = b_in.reshape(1, H)
    bm = b_msg.reshape(1, H)

    af = edge_attr.reshape(E * D_EDGE_)
    attr2 = jnp.pad(af, (0, (E_PAD - E) * D_EDGE_)).reshape(
        E_PAD // 2, 2 * D_EDGE_)
    w2 = jnp.zeros((2 * D_EDGE_, 2 * H), jnp.float32)
    w2 = w2.at[:D_EDGE_, :H].set(W_edge).at[D_EDGE_:, H:].set(W_edge)
    be2 = jnp.tile(b_edge, 2).reshape(1, 2 * H)

    e_pad = pl.pallas_call(
        _e_kernel,
        grid=(E_PAD // 2 // EBLK,),
        in_specs=[pl.BlockSpec((EBLK, 2 * D_EDGE_), lambda i: (i, 0)),
                  pl.BlockSpec((2 * D_EDGE_, 2 * H), lambda i: (0, 0)),
                  pl.BlockSpec((1, 2 * H), lambda i: (0, 0))],
        out_specs=pl.BlockSpec((EBLK, 2 * H), lambda i: (i, 0)),
        out_shape=jax.ShapeDtypeStruct((E_PAD // 2, 2 * H), jnp.float32),
    )(attr2, w2, be2)

    degp = _sc_deg(dst_p, zeros_d)

    GRID_N = N // NBLK
    xm, h0, deg_b = pl.pallas_call(
        _pre_kernel,
        grid=(GRID_N,),
        in_specs=[pl.BlockSpec((NBLK, D_IN), lambda i: (i, 0)),
                  pl.BlockSpec((NBLK, DW), lambda i: (i, 0)),
                  pl.BlockSpec((NBLK, DW), lambda i: (i + GRID_N, 0)),
                  pl.BlockSpec((D_IN, H), lambda i: (0, 0)),
                  pl.BlockSpec((1, H), lambda i: (0, 0)),
                  pl.BlockSpec((1, H), lambda i: (0, 0))],
        out_specs=[pl.BlockSpec((NBLK, D_IN), lambda i: (i, 0)),
                   pl.BlockSpec((NBLK, H), lambda i: (i, 0)),
                   pl.BlockSpec((NBLK, H), lambda i: (i, 0))],
        out_shape=[jax.ShapeDtypeStruct((N, D_IN), jnp.float32),
                   jax.ShapeDtypeStruct((N, H), jnp.float32),
                   jax.ShapeDtypeStruct((N, H), jnp.float32)],
    )(x, degp, degp, W_in, bi, bm)

    update = pl.pallas_call(
        _update_kernel,
        grid=(GRID_N,),
        in_specs=[pl.BlockSpec((NBLK, 2 * H), lambda i: (i, 0)),
                  pl.BlockSpec((NBLK, H), lambda i: (i, 0)),
                  pl.BlockSpec((NBLK, H), lambda i: (i, 0)),
                  pl.BlockSpec((H, H), lambda i: (0, 0))],
        out_specs=pl.BlockSpec((NBLK, H), lambda i: (i, 0)),
        out_shape=jax.ShapeDtypeStruct((N, H), jnp.float32),
    )

    h = h0
    layers = [h0]
    for _ in range(L):
        S = _sc_layer(h, e_pad, src_p, dst_p)
        h = update(S, h, deg_b, W_msg)
        layers.append(h)

    ge = pl.pallas_call(
        _max_kernel,
        grid=(GRID_N,),
        in_specs=[pl.BlockSpec((NBLK, D_IN), lambda i: (i, 0))] +
                 [pl.BlockSpec((NBLK, H), lambda i: (i, 0))] * 7,
        out_specs=pl.BlockSpec((1, F), lambda i: (0, 0)),
        out_shape=jax.ShapeDtypeStruct((1, F), jnp.float32),
    )(xm, *layers)

    out = pl.pallas_call(
        _assemble_kernel,
        grid=(GRID_N,),
        in_specs=[pl.BlockSpec((NBLK, D_IN), lambda i: (i, 0))] +
                 [pl.BlockSpec((NBLK, H), lambda i: (i, 0))] * 7 +
                 [pl.BlockSpec((1, F), lambda i: (0, 0))],
        out_specs=pl.BlockSpec((NBLK, 2 * F), lambda i: (i, 0)),
        out_shape=jax.ShapeDtypeStruct((N, 2 * F), jnp.float32),
    )(xm, *layers, ge)
    return out


# speedup vs baseline: 1.0457x; 1.0457x over previous
import jax
import jax.numpy as jnp
from jax import lax
from jax.experimental import pallas as pl
from jax.experimental.pallas import tpu as pltpu
from jax.experimental.pallas import tpu_sc as plsc

N = 10000
E = 320000
D_IN = 16
D_EDGE_ = 4
H = 64
L = 6
F = D_IN + H * (L + 1)

NC, NS = 2, 16
NW = NC * NS
C = 128
NCHUNK = 80
EPW = NCHUNK * C
E_PAD = NW * EPW
NP = 10112
RZ = NP // NS
RO = 632
RO_LAST = N - 15 * RO
DW = 16


SUP = C
NCH_TOT = E_PAD // C
K0 = 136
K1 = 24


def _sc_layer_body(h_hbm, e_hbm, src_hbm, dst_hbm, out_hbm,
                   src_all, dst_all, gbuf, ebuf, sbuf,
                   sem_g0, sem_g1, sem_e0, sem_e1, sem_s0, sem_s1, acc):
    cid = lax.axis_index("c")
    sid = lax.axis_index("s")
    sem_g = (sem_g0, sem_g1)
    sem_e = (sem_e0, sem_e1)
    sem_s = (sem_s0, sem_s1)
    gbase = jnp.where(cid == 0, sid * K0, NS * K0 + sid * K1)
    nk = jnp.where(cid == 0, K0, K1)

    @pl.when(cid == 0)
    def _stage0():
        pltpu.sync_copy(src_hbm.at[pl.ds(sid * K0, K0)],
                        src_all.at[pl.ds(0, K0)])
        pltpu.sync_copy(dst_hbm.at[pl.ds(sid * K0, K0)],
                        dst_all.at[pl.ds(0, K0)])

    @pl.when(cid == 1)
    def _stage1():
        pltpu.sync_copy(src_hbm.at[pl.ds(NS * K0 + sid * K1, K1)],
                        src_all.at[pl.ds(0, K1)])
        pltpu.sync_copy(dst_hbm.at[pl.ds(NS * K0 + sid * K1, K1)],
                        dst_all.at[pl.ds(0, K1)])

    @plsc.parallel_loop(0, SUP, unroll=8)
    def _zrow(r):
        for c4 in range(4):
            sbuf[0, r, pl.ds(c4 * 16, 16)] = jnp.zeros((16,), jnp.float32)

    for i in range(4):
        pltpu.sync_copy(sbuf.at[0],
                        acc.at[pl.ds(sid * RZ + i * SUP, SUP)])
    pltpu.sync_copy(sbuf.at[0, pl.ds(0, RZ - 4 * SUP)],
                    acc.at[pl.ds(sid * RZ + 4 * SUP, RZ - 4 * SUP)])
    plsc.subcore_barrier()

    def issue_loads(k, b):
        pltpu.async_copy(h_hbm.at[src_all.at[k]], gbuf.at[b], sem_g[b])
        pltpu.async_copy(e_hbm.at[pl.ds((gbase + k) * (C // 2), C // 2)],
                         ebuf.at[b], sem_e[b])

    def wait_f32(sem, dst):
        pltpu.make_async_copy(h_hbm.at[pl.ds(0, SUP)], dst, sem).wait()

    def wait_e(sem, dst):
        pltpu.make_async_copy(e_hbm.at[pl.ds(0, SUP // 2)], dst, sem).wait()

    def do_superchunk(k, b):
        @pl.when(k + 1 < nk)
        def _pre():
            issue_loads(k + 1, 1 - b)

        @pl.when(k >= 2)
        def _drain():
            wait_f32(sem_s[b], sbuf.at[b])

        wait_f32(sem_g[b], gbuf.at[b])
        wait_e(sem_e[b], ebuf.at[b])

        @plsc.parallel_loop(0, SUP // 2, unroll=4)
        def _row(r2):
            for c8 in range(8):
                r = 2 * r2 + c8 // 4
                sl = pl.ds((c8 % 4) * 16, 16)
                sbuf[b, r, sl] = jnp.maximum(
                    gbuf[b, r, sl] + ebuf[b, r2, pl.ds(c8 * 16, 16)], 0.0)

        pltpu.async_copy(sbuf.at[b], acc.at[dst_all.at[k]], sem_s[b],
                         add=True)

    issue_loads(0, 0)

    def pair(j, carry):
        do_superchunk(2 * j, 0)
        do_superchunk(2 * j + 1, 1)
        return carry

    lax.fori_loop(0, nk // 2, pair, 0)
    wait_f32(sem_s[0], sbuf.at[0])
    wait_f32(sem_s[1], sbuf.at[1])
    plsc.subcore_barrier()

    @pl.when(sid < NS - 1)
    def _full():
        pltpu.sync_copy(acc.at[pl.ds(sid * RO, RO)],
                        out_hbm.at[pl.ds(cid * N + sid * RO, RO)])

    @pl.when(sid == NS - 1)
    def _last():
        pltpu.sync_copy(acc.at[pl.ds((NS - 1) * RO, RO_LAST)],
                        out_hbm.at[pl.ds(cid * N + (NS - 1) * RO, RO_LAST)])


_sc_layer = pl.kernel(
    _sc_layer_body,
    out_type=jax.ShapeDtypeStruct((2 * N, H), jnp.float32),
    mesh=plsc.VectorSubcoreMesh(core_axis_name="c", subcore_axis_name="s"),
    scratch_types=[
        pltpu.VMEM((K0, C), jnp.int32),
        pltpu.VMEM((K0, C), jnp.int32),
        pltpu.VMEM((2, SUP, H), jnp.float32),
        pltpu.VMEM((2, SUP // 2, 2 * H), jnp.float32),
        pltpu.VMEM((2, SUP, H), jnp.float32),
        pltpu.SemaphoreType.DMA,
        pltpu.SemaphoreType.DMA,
        pltpu.SemaphoreType.DMA,
        pltpu.SemaphoreType.DMA,
        pltpu.SemaphoreType.DMA,
        pltpu.SemaphoreType.DMA,
        pltpu.VMEM_SHARED((NP, H), jnp.float32),
    ],
    compiler_params=pltpu.CompilerParams(use_tc_tiling_on_sc=False),
)


def _sc_deg_body(dst_hbm, zero_hbm, out_hbm, dst_all, obuf, acc):
    cid = lax.axis_index("c")
    sid = lax.axis_index("s")
    wid = sid * NC + cid
    pltpu.sync_copy(dst_hbm.at[pl.ds(wid * NCHUNK, NCHUNK)], dst_all)
    pltpu.sync_copy(zero_hbm.at[pl.ds(sid * RZ, RZ)],
                    acc.at[pl.ds(sid * RZ, RZ)])

    def orow(r, carry):
        obuf[r, pl.ds(0, 16)] = jnp.full((16,), 1.0, jnp.float32)
        return carry
    lax.fori_loop(0, C, orow, 0)
    plsc.subcore_barrier()

    def chunk(k, carry):
        pltpu.sync_copy(obuf, acc.at[dst_all.at[k]], add=True)
        return carry
    lax.fori_loop(0, NCHUNK, chunk, 0)
    plsc.subcore_barrier()

    @pl.when(sid < NS - 1)
    def _full():
        pltpu.sync_copy(acc.at[pl.ds(sid * RO, RO)],
                        out_hbm.at[pl.ds(cid * N + sid * RO, RO)])

    @pl.when(sid == NS - 1)
    def _last():
        pltpu.sync_copy(acc.at[pl.ds((NS - 1) * RO, RO_LAST)],
                        out_hbm.at[pl.ds(cid * N + (NS - 1) * RO, RO_LAST)])


_sc_deg = pl.kernel(
    _sc_deg_body,
    out_type=jax.ShapeDtypeStruct((2 * N, DW), jnp.float32),
    mesh=plsc.VectorSubcoreMesh(core_axis_name="c", subcore_axis_name="s"),
    scratch_types=[
        pltpu.VMEM((NCHUNK, C), jnp.int32),
        pltpu.VMEM((C, DW), jnp.float32),
        pltpu.VMEM_SHARED((NP, DW), jnp.float32),
    ],
    compiler_params=pltpu.CompilerParams(use_tc_tiling_on_sc=False),
)


EBLK = 4096
NBLK = 2000


def _e_kernel(attr_ref, we_ref, be_ref, out_ref):
    out_ref[...] = attr_ref[...] @ we_ref[...] + be_ref[...]


def _pre_kernel(x_ref, d0_ref, d1_ref, win_ref, bin_ref, bmsg_ref,
                xm_ref, h0_ref, degb_ref):
    col = lax.broadcasted_iota(jnp.int32, (NBLK, D_IN), 1)
    keep = (col < 2) | ((col >= 4) & (col < 10))
    xm = jnp.where(keep, x_ref[...], 0.0)
    xm_ref[...] = xm
    h0_ref[...] = xm @ win_ref[...] + bin_ref[...]
    deg = d0_ref[:, 0:1] + d1_ref[:, 0:1]
    degb_ref[...] = deg * bmsg_ref[...]


def _update_kernel(s0_ref, s1_ref, h_ref, degb_ref, w_ref, out_ref):
    out_ref[...] = h_ref[...] + (s0_ref[...] + s1_ref[...]) @ w_ref[...] \
        + degb_ref[...]


def _max_kernel(xm_ref, h0, h1, h2, h3, h4, h5, h6, out_ref):
    i = pl.program_id(0)

    @pl.when(i == 0)
    def _init():
        out_ref[...] = jnp.full_like(out_ref, -jnp.inf)

    hs = [h0, h1, h2, h3, h4, h5, h6]
    out_ref[:, 0:D_IN] = jnp.maximum(
        out_ref[:, 0:D_IN], jnp.max(xm_ref[...], axis=0, keepdims=True))
    for j, hr in enumerate(hs):
        sl = slice(D_IN + j * H, D_IN + (j + 1) * H)
        out_ref[:, sl] = jnp.maximum(
            out_ref[:, sl], jnp.max(hr[...], axis=0, keepdims=True))


def _assemble_kernel(xm_ref, h0, h1, h2, h3, h4, h5, h6, ge_ref, out_ref):
    hs = [h0, h1, h2, h3, h4, h5, h6]
    out_ref[:, 0:D_IN] = xm_ref[...]
    for j, hr in enumerate(hs):
        out_ref[:, D_IN + j * H:D_IN + (j + 1) * H] = hr[...]
    out_ref[:, F:] = jnp.broadcast_to(ge_ref[...], (NBLK, F))


def kernel(x, edge_index, edge_attr, W_edge, b_edge, W_in, b_in, W_msg, b_msg):
    src = edge_index[0]
    dst = edge_index[1]
    src_p = jnp.pad(src, (0, E_PAD - E)).reshape(NCH_TOT, C)
    dst_p = jnp.pad(dst, (0, E_PAD - E),
                    constant_values=N).reshape(NCH_TOT, C)
    zeros_d = jnp.zeros((NP, DW), jnp.float32)

    bi = b_in.reshape(1, H)
    bm = b_msg.reshape(1, H)

    af = edge_attr.reshape(E * D_EDGE_)
    attr2 = jnp.pad(af, (0, (E_PAD - E) * D_EDGE_)).reshape(
        E_PAD // 2, 2 * D_EDGE_)
    w2 = jnp.zeros((2 * D_EDGE_, 2 * H), jnp.float32)
    w2 = w2.at[:D_EDGE_, :H].set(W_edge).at[D_EDGE_:, H:].set(W_edge)
    be2 = jnp.tile(b_edge, 2).reshape(1, 2 * H)

    e_pad = pl.pallas_call(
        _e_kernel,
        grid=(E_PAD // 2 // EBLK,),
        in_specs=[pl.BlockSpec((EBLK, 2 * D_EDGE_), lambda i: (i, 0)),
                  pl.BlockSpec((2 * D_EDGE_, 2 * H), lambda i: (0, 0)),
                  pl.BlockSpec((1, 2 * H), lambda i: (0, 0))],
        out_specs=pl.BlockSpec((EBLK, 2 * H), lambda i: (i, 0)),
        out_shape=jax.ShapeDtypeStruct((E_PAD // 2, 2 * H), jnp.float32),
    )(attr2, w2, be2)

    degp = _sc_deg(dst_p, zeros_d)

    GRID_N = N // NBLK
    xm, h0, deg_b = pl.pallas_call(
        _pre_kernel,
        grid=(GRID_N,),
        in_specs=[pl.BlockSpec((NBLK, D_IN), lambda i: (i, 0)),
                  pl.BlockSpec((NBLK, DW), lambda i: (i, 0)),
                  pl.BlockSpec((NBLK, DW), lambda i: (i + GRID_N, 0)),
                  pl.BlockSpec((D_IN, H), lambda i: (0, 0)),
                  pl.BlockSpec((1, H), lambda i: (0, 0)),
                  pl.BlockSpec((1, H), lambda i: (0, 0))],
        out_specs=[pl.BlockSpec((NBLK, D_IN), lambda i: (i, 0)),
                   pl.BlockSpec((NBLK, H), lambda i: (i, 0)),
                   pl.BlockSpec((NBLK, H), lambda i: (i, 0))],
        out_shape=[jax.ShapeDtypeStruct((N, D_IN), jnp.float32),
                   jax.ShapeDtypeStruct((N, H), jnp.float32),
                   jax.ShapeDtypeStruct((N, H), jnp.float32)],
    )(x, degp, degp, W_in, bi, bm)

    update = pl.pallas_call(
        _update_kernel,
        grid=(GRID_N,),
        in_specs=[pl.BlockSpec((NBLK, H), lambda i: (i, 0)),
                  pl.BlockSpec((NBLK, H), lambda i: (i + GRID_N, 0)),
                  pl.BlockSpec((NBLK, H), lambda i: (i, 0)),
                  pl.BlockSpec((NBLK, H), lambda i: (i, 0)),
                  pl.BlockSpec((H, H), lambda i: (0, 0))],
        out_specs=pl.BlockSpec((NBLK, H), lambda i: (i, 0)),
        out_shape=jax.ShapeDtypeStruct((N, H), jnp.float32),
    )

    h = h0
    layers = [h0]
    for _ in range(L):
        S = _sc_layer(h, e_pad, src_p, dst_p)
        h = update(S, S, h, deg_b, W_msg)
        layers.append(h)

    ge = pl.pallas_call(
        _max_kernel,
        grid=(GRID_N,),
        in_specs=[pl.BlockSpec((NBLK, D_IN), lambda i: (i, 0))] +
                 [pl.BlockSpec((NBLK, H), lambda i: (i, 0))] * 7,
        out_specs=pl.BlockSpec((1, F), lambda i: (0, 0)),
        out_shape=jax.ShapeDtypeStruct((1, F), jnp.float32),
    )(xm, *layers)

    out = pl.pallas_call(
        _assemble_kernel,
        grid=(GRID_N,),
        in_specs=[pl.BlockSpec((NBLK, D_IN), lambda i: (i, 0))] +
                 [pl.BlockSpec((NBLK, H), lambda i: (i, 0))] * 7 +
                 [pl.BlockSpec((1, F), lambda i: (0, 0))],
        out_specs=pl.BlockSpec((NBLK, 2 * F), lambda i: (i, 0)),
        out_shape=jax.ShapeDtypeStruct((N, 2 * F), jnp.float32),
    )(xm, *layers, ge)
    return out
